# baseline (speedup 1.0000x reference)
import functools
import math

import jax
import jax.numpy as jnp
from jax import lax
from jax.experimental import pallas as pl
from jax.experimental.pallas import tpu as pltpu
from jax.experimental.pallas import tpu_sc as plsc

D_MODEL = 64
SCALE = math.sqrt(D_MODEL)

NC, NS = 2, 16
NW = NC * NS
LANES = 16

ROWS_PER_GATHER = 128
GATHERS_PER_STEP = 4
STEP_ROWS = ROWS_PER_GATHER * GATHERS_PER_STEP


@functools.cache
def _build(n_idx):
    assert n_idx % (NW * STEP_ROWS) == 0
    b_per_w = n_idx // NW
    idx_rows_per_w = b_per_w // ROWS_PER_GATHER
    n_steps = idx_rows_per_w // GATHERS_PER_STEP

    mesh = plsc.VectorSubcoreMesh(core_axis_name="c", subcore_axis_name="s")

    @functools.partial(
        pl.kernel,
        out_type=jax.ShapeDtypeStruct((n_idx, D_MODEL), jnp.float32),
        mesh=mesh,
        scratch_types=[
            pltpu.VMEM((idx_rows_per_w, ROWS_PER_GATHER), jnp.int32),
            pltpu.VMEM((STEP_ROWS, D_MODEL), jnp.float32),
            pltpu.SemaphoreType.DMA,
        ],
        compiler_params=pltpu.CompilerParams(use_tc_tiling_on_sc=False),
    )
    def emb_kernel(idx_hbm, lut_hbm, out_hbm, idx_v, rows_v, sem):
        wid = lax.axis_index("s") * NC + lax.axis_index("c")
        row_base = wid * idx_rows_per_w
        out_base = wid * b_per_w

        pltpu.sync_copy(idx_hbm.at[pl.ds(row_base, idx_rows_per_w)], idx_v)

        def step(g, carry):
            copies = []
            for j in range(GATHERS_PER_STEP):
                r = g * GATHERS_PER_STEP + j
                copies.append(pltpu.async_copy(
                    lut_hbm.at[idx_v.at[r]],
                    rows_v.at[pl.ds(j * ROWS_PER_GATHER, ROWS_PER_GATHER)],
                    sem,
                ))
            for cp in copies:
                cp.wait()

            def scale_row(i, c2):
                for c in range(D_MODEL // LANES):
                    sl = (i, pl.ds(c * LANES, LANES))
                    rows_v[sl] = rows_v[sl] * SCALE
                return c2
            lax.fori_loop(0, STEP_ROWS, scale_row, 0)

            pltpu.sync_copy(
                rows_v, out_hbm.at[pl.ds(out_base + g * STEP_ROWS, STEP_ROWS)])
            return carry

        lax.fori_loop(0, n_steps, step, 0)

    return emb_kernel


def kernel(x, lut):
    b, s = x.shape
    n = b * s
    idx = x.reshape(n // ROWS_PER_GATHER, ROWS_PER_GATHER).astype(jnp.int32)
    out = _build(n)(idx, lut)
    return out.reshape(b, s, D_MODEL)


# speedup vs baseline: 1.0415x; 1.0415x over previous
import functools
import math

import jax
import jax.numpy as jnp
from jax import lax
from jax.experimental import pallas as pl
from jax.experimental.pallas import tpu as pltpu
from jax.experimental.pallas import tpu_sc as plsc

D_MODEL = 64
SCALE = math.sqrt(D_MODEL)

NC, NS = 2, 16
NW = NC * NS
LANES = 16

ROWS_PER_GATHER = 128
GATHERS_PER_STEP = 4
STEP_ROWS = ROWS_PER_GATHER * GATHERS_PER_STEP


@functools.cache
def _build(n_idx):
    assert n_idx % (NW * STEP_ROWS) == 0
    b_per_w = n_idx // NW
    idx_rows_per_w = b_per_w // ROWS_PER_GATHER
    n_steps = idx_rows_per_w // GATHERS_PER_STEP

    mesh = plsc.VectorSubcoreMesh(core_axis_name="c", subcore_axis_name="s")

    @functools.partial(
        pl.kernel,
        out_type=jax.ShapeDtypeStruct((n_idx, D_MODEL), jnp.float32),
        mesh=mesh,
        scratch_types=[
            pltpu.VMEM((idx_rows_per_w, ROWS_PER_GATHER), jnp.int32),
            pltpu.VMEM((STEP_ROWS, D_MODEL), jnp.float32),
            pltpu.SemaphoreType.DMA,
        ],
        compiler_params=pltpu.CompilerParams(use_tc_tiling_on_sc=False),
    )
    def emb_kernel(idx_hbm, lut_hbm, out_hbm, idx_v, rows_v, sem):
        wid = lax.axis_index("s") * NC + lax.axis_index("c")
        row_base = wid * idx_rows_per_w
        out_base = wid * b_per_w

        pltpu.sync_copy(idx_hbm.at[pl.ds(row_base, idx_rows_per_w)], idx_v)

        def step(g, carry):
            copies = []
            for j in range(GATHERS_PER_STEP):
                r = g * GATHERS_PER_STEP + j
                copies.append(pltpu.async_copy(
                    lut_hbm.at[idx_v.at[r]],
                    rows_v.at[pl.ds(j * ROWS_PER_GATHER, ROWS_PER_GATHER)],
                    sem,
                ))
            for cp in copies:
                cp.wait()

            @plsc.parallel_loop(0, STEP_ROWS, unroll=8)
            def scale_row(i):
                for c in range(D_MODEL // LANES):
                    sl = (i, pl.ds(c * LANES, LANES))
                    rows_v[sl] = rows_v[sl] * SCALE

            pltpu.sync_copy(
                rows_v, out_hbm.at[pl.ds(out_base + g * STEP_ROWS, STEP_ROWS)])
            return carry

        lax.fori_loop(0, n_steps, step, 0)

    return emb_kernel


def kernel(x, lut):
    b, s = x.shape
    n = b * s
    idx = x.reshape(n // ROWS_PER_GATHER, ROWS_PER_GATHER).astype(jnp.int32)
    out = _build(n)(idx, lut)
    return out.reshape(b, s, D_MODEL)


# speedup vs baseline: 1.1184x; 1.0738x over previous
import functools
import math

import jax
import jax.numpy as jnp
from jax import lax
from jax.experimental import pallas as pl
from jax.experimental.pallas import tpu as pltpu
from jax.experimental.pallas import tpu_sc as plsc

D_MODEL = 64
SCALE = math.sqrt(D_MODEL)

NC, NS = 2, 16
NW = NC * NS
LANES = 16

CHUNK = 256
NBUF = 4


@functools.cache
def _build(n_idx):
    assert n_idx % (NW * CHUNK * NBUF) == 0
    b_per_w = n_idx // NW
    n_steps = b_per_w // CHUNK

    mesh = plsc.VectorSubcoreMesh(core_axis_name="c", subcore_axis_name="s")

    @functools.partial(
        pl.kernel,
        out_type=jax.ShapeDtypeStruct((n_idx, D_MODEL), jnp.float32),
        mesh=mesh,
        scratch_types=[
            pltpu.VMEM((b_per_w,), jnp.int32),
            pltpu.VMEM((NBUF * CHUNK, D_MODEL), jnp.float32),
            pltpu.SemaphoreType.DMA,
            pltpu.SemaphoreType.DMA,
        ],
        compiler_params=pltpu.CompilerParams(use_tc_tiling_on_sc=False),
    )
    def emb_kernel(idx_hbm, lut_hbm, out_hbm, idx_v, rows_v, gsem, wsem):
        wid = lax.axis_index("s") * NC + lax.axis_index("c")
        base = wid * b_per_w

        pltpu.sync_copy(idx_hbm.at[pl.ds(base, b_per_w)], idx_v)

        def buf(j):
            return rows_v.at[pl.ds(j * CHUNK, CHUNK)]

        def fire_gather(s, j):
            pltpu.async_copy(
                lut_hbm.at[idx_v.at[pl.ds(pl.multiple_of(s * CHUNK, CHUNK),
                                          CHUNK)]],
                buf(j), gsem)

        fire_gather(0, 0)
        fire_gather(1, 1)

        @pl.loop(0, n_steps, step=NBUF)
        def outer(g):
            for b in range(NBUF):
                s = g + b
                pltpu.make_async_copy(
                    lut_hbm.at[pl.ds(0, CHUNK)], buf(b), gsem).wait()

                @plsc.parallel_loop(0, CHUNK, unroll=8)
                def scale_row(i):
                    for c in range(D_MODEL // LANES):
                        sl = (b * CHUNK + i, pl.ds(c * LANES, LANES))
                        rows_v[sl] = rows_v[sl] * SCALE

                pltpu.async_copy(
                    buf(b),
                    out_hbm.at[pl.ds(pl.multiple_of(base + s * CHUNK, CHUNK),
                                     CHUNK)],
                    wsem)

                @pl.when(s >= 2)
                def _():
                    pltpu.make_async_copy(
                        buf((b + 2) % NBUF), out_hbm.at[pl.ds(0, CHUNK)],
                        wsem).wait()

                @pl.when(s + 2 < n_steps)
                def _():
                    fire_gather(s + 2, (b + 2) % NBUF)

        for _ in range(2):
            pltpu.make_async_copy(
                buf(0), out_hbm.at[pl.ds(0, CHUNK)], wsem).wait()

    return emb_kernel


def kernel(x, lut):
    b, s = x.shape
    n = b * s
    idx = x.reshape(n).astype(jnp.int32)
    out = _build(n)(idx, lut)
    return out.reshape(b, s, D_MODEL)
